# baseline (speedup 1.0000x reference)
import functools

import jax
import jax.numpy as jnp
from jax import lax
from jax.experimental import pallas as pl
from jax.experimental.pallas import tpu as pltpu
from jax.experimental.pallas import tpu_sc as plsc

_N = 10000
_E = 320000
_D = 128
_DE = 16
_G = 64

_NC = 2
_NS = 16
_NW = _NC * _NS
_EPW = _E // _NW
_CHUNK = 100
_NCHUNK = _EPW // _CHUNK
_NB = 2
_BLK = 50
_NBLK = _NCHUNK // _BLK
_NP = 10112
_RPT = _NP // _NS


def _sc_edge_pass(with_ea):
    out_type = [jax.ShapeDtypeStruct((_NC, _NP, _D), jnp.float32)]
    scratch = [
        pltpu.VMEM((_BLK, _CHUNK), jnp.int32),
        pltpu.VMEM((_BLK, _CHUNK), jnp.int32),
        [pltpu.VMEM((_CHUNK, _D), jnp.float32) for _ in range(_NB)],
        pltpu.VMEM_SHARED((_NP, _D), jnp.float32),
        [pltpu.SemaphoreType.DMA for _ in range(_NB)],
    ]
    if with_ea:
        out_type.append(jax.ShapeDtypeStruct((_NC, _NP, _DE), jnp.float32))
        scratch += [
            [pltpu.VMEM((_CHUNK, _DE), jnp.float32) for _ in range(_NB)],
            pltpu.VMEM_SHARED((_NP, _DE), jnp.float32),
            [pltpu.SemaphoreType.DMA for _ in range(_NB)],
        ]

    mesh = plsc.VectorSubcoreMesh(core_axis_name="c", subcore_axis_name="s")

    def body(*refs):
        if with_ea:
            (x_hbm, src_hbm, dst_hbm, ea_hbm, zx_hbm, zea_hbm,
             acc_out, ea_out,
             sidx_v, didx_v, rows_v, acc_sp, gsem, ea_v, ea_sp, esem) = refs
        else:
            (x_hbm, src_hbm, dst_hbm, zx_hbm,
             acc_out,
             sidx_v, didx_v, rows_v, acc_sp, gsem) = refs

        cid = lax.axis_index("c")
        sid = lax.axis_index("s")
        wid = sid * _NC + cid

        pltpu.sync_copy(zx_hbm, acc_sp.at[pl.ds(sid * _RPT, _RPT)])
        if with_ea:
            pltpu.sync_copy(zea_hbm, ea_sp.at[pl.ds(sid * _RPT, _RPT)])
        plsc.subcore_barrier()

        def run_block(blk):
            pltpu.sync_copy(src_hbm.at[wid, pl.ds(blk * _BLK, _BLK)], sidx_v)
            pltpu.sync_copy(dst_hbm.at[wid, pl.ds(blk * _BLK, _BLK)], didx_v)

            def issue(r, b):
                pltpu.async_copy(x_hbm.at[sidx_v.at[r]], rows_v[b], gsem[b])
                if with_ea:
                    pltpu.async_copy(ea_hbm.at[wid, blk * _BLK + r],
                                     ea_v[b], esem[b])

            for b in range(_NB):
                issue(b, b)

            def outer(rr, carry):
                for b in range(_NB):
                    r = rr * _NB + b
                    pltpu.make_async_copy(
                        x_hbm.at[sidx_v.at[r]], rows_v[b], gsem[b]).wait()
                    pltpu.sync_copy(rows_v[b], acc_sp.at[didx_v.at[r]],
                                    add=True)
                    if with_ea:
                        pltpu.make_async_copy(
                            ea_hbm.at[wid, blk * _BLK + r],
                            ea_v[b], esem[b]).wait()
                        pltpu.sync_copy(ea_v[b], ea_sp.at[didx_v.at[r]],
                                        add=True)

                    @pl.when(r + _NB < _BLK)
                    def _():
                        issue(r + _NB, b)
                return carry

            lax.fori_loop(0, _BLK // _NB, outer, 0)

        for blk in range(_NBLK):
            run_block(blk)
        plsc.subcore_barrier()

        pltpu.sync_copy(acc_sp.at[pl.ds(sid * _RPT, _RPT)],
                        acc_out.at[cid, pl.ds(sid * _RPT, _RPT)])
        if with_ea:
            pltpu.sync_copy(ea_sp.at[pl.ds(sid * _RPT, _RPT)],
                            ea_out.at[cid, pl.ds(sid * _RPT, _RPT)])

    return pl.kernel(
        body, mesh=mesh, out_type=out_type, scratch_types=scratch,
        compiler_params=pltpu.CompilerParams(use_tc_tiling_on_sc=False))


@functools.cache
def _sc_pass(with_ea):
    return _sc_edge_pass(with_ea)


_TCOLS = 2560


def _tc_ea_t_body(eat_ref, out_ref):
    eye = jnp.eye(_DE, dtype=jnp.float32)
    out_ref[...] = lax.dot_general(eat_ref[...], eye, (((0,), (0,)), ((), ())))


_tc_ea_t = pl.pallas_call(
    _tc_ea_t_body,
    grid=(_E // _TCOLS,),
    in_specs=[pl.BlockSpec((_DE, _TCOLS), lambda i: (0, i))],
    out_specs=pl.BlockSpec((_TCOLS, _DE), lambda i: (i, 0)),
    out_shape=jax.ShapeDtypeStruct((_E, _DE), jnp.float32),
)


def _onehot_and_invcnt(batch2d):
    gids = lax.broadcasted_iota(jnp.int32, (1, _G), 1)
    onehot = (batch2d == gids).astype(jnp.float32)
    ones = jnp.ones((_N, 1), jnp.float32)
    cnt = lax.dot_general(onehot, ones, (((0,), (0,)), ((), ())))
    return onehot, 1.0 / jnp.maximum(cnt, 1.0)


def _tc_conv1_body(x_ref, s1_ref, batch_ref, w1_ref, be_ref,
                   out_ref, outb_ref):
    x = x_ref[...]
    out = x + s1_ref[0, :_N] + s1_ref[1, :_N]
    onehot, invcnt = _onehot_and_invcnt(batch_ref[...])
    pooled = lax.dot_general(onehot, out, (((0,), (0,)), ((), ())))
    vmsg = pooled * invcnt
    out = out + lax.dot_general(onehot, vmsg, (((1,), (0,)), ((), ())))
    out = jnp.maximum(lax.dot_general(out, w1_ref[...],
                                      (((1,), (0,)), ((), ()))), 0.0)
    x1 = out + x
    out_ref[...] = x1
    outb_ref[...] = x1 + be_ref[...]


_tc_conv1 = pl.pallas_call(
    _tc_conv1_body,
    out_shape=[jax.ShapeDtypeStruct((_N, _D), jnp.float32),
               jax.ShapeDtypeStruct((_N, _D), jnp.float32)],
)


def _tc_conv2_body(x1_ref, s2_ref, ea_ref, batch_ref, we_ref,
                   wn1_ref, bn1_ref, wn2_ref, bn2_ref, wfc_ref, bfc_ref,
                   out_ref):
    x1 = x1_ref[...]
    ea = ea_ref[0, :_N] + ea_ref[1, :_N]
    agg = s2_ref[0, :_N] + s2_ref[1, :_N] + lax.dot_general(
        ea, we_ref[...], (((1,), (0,)), ((), ())))
    h = jnp.maximum(lax.dot_general(agg, wn1_ref[...],
                                    (((1,), (0,)), ((), ()))) + bn1_ref[...],
                    0.0)
    out2 = lax.dot_general(h, wn2_ref[...],
                           (((1,), (0,)), ((), ()))) + bn2_ref[...]
    x2 = out2 + x1
    onehot, invcnt = _onehot_and_invcnt(batch_ref[...])
    pooled = lax.dot_general(onehot, x2, (((0,), (0,)), ((), ()))) * invcnt
    out_ref[...] = lax.dot_general(pooled, wfc_ref[...],
                                   (((1,), (0,)), ((), ()))) + bfc_ref[...]


_tc_conv2 = pl.pallas_call(
    _tc_conv2_body,
    out_shape=jax.ShapeDtypeStruct((_G, _D), jnp.float32),
)


def kernel(x, edge_index, edge_attr, batch, W1, We, be, Wn1, bn1, Wn2, bn2,
           Wfc, bfc):
    src = edge_index[0].reshape(_NW, _NCHUNK, _CHUNK)
    dst = edge_index[1].reshape(_NW, _NCHUNK, _CHUNK)
    ea = _tc_ea_t(edge_attr.T).reshape(_NW, _NCHUNK, _CHUNK, _DE)
    zx = jnp.zeros((_RPT, _D), jnp.float32)
    zea = jnp.zeros((_RPT, _DE), jnp.float32)
    batch2d = batch[:, None]

    s1p, eap = _sc_pass(True)(x, src, dst, ea, zx, zea)
    x1, x1b = _tc_conv1(x, s1p, batch2d, W1, be[None, :])
    s2p = _sc_pass(False)(x1b, src, dst, zx)
    if isinstance(s2p, (list, tuple)):
        (s2p,) = s2p
    return _tc_conv2(x1, s2p, eap, batch2d, We,
                     Wn1, bn1[None, :], Wn2, bn2[None, :], Wfc, bfc[None, :])


# speedup vs baseline: 1.0685x; 1.0685x over previous
import functools

import jax
import jax.numpy as jnp
from jax import lax
from jax.experimental import pallas as pl
from jax.experimental.pallas import tpu as pltpu
from jax.experimental.pallas import tpu_sc as plsc

_N = 10000
_E = 320000
_D = 128
_DE = 16
_G = 64

_NC = 2
_NS = 16
_NW = _NC * _NS
_EPW = _E // _NW
_CHUNK = 50
_NCHUNK = _EPW // _CHUNK
_NB = 4
_BLK = 40
_NBLK = _NCHUNK // _BLK
_NP = 10112
_RPT = _NP // _NS


def _sc_edge_pass(with_ea):
    out_type = [jax.ShapeDtypeStruct((_NC, _NP, _D), jnp.float32)]
    scratch = [
        pltpu.VMEM((_BLK, _CHUNK), jnp.int32),
        pltpu.VMEM((_BLK, _CHUNK), jnp.int32),
        [pltpu.VMEM((_CHUNK, _D), jnp.float32) for _ in range(_NB)],
        pltpu.VMEM_SHARED((_NP, _D), jnp.float32),
        [pltpu.SemaphoreType.DMA for _ in range(_NB)],
        [pltpu.SemaphoreType.DMA for _ in range(_NB)],
    ]
    if with_ea:
        out_type.append(jax.ShapeDtypeStruct((_NC, _NP, _DE), jnp.float32))
        scratch += [
            [pltpu.VMEM((_CHUNK, _DE), jnp.float32) for _ in range(_NB)],
            pltpu.VMEM_SHARED((_NP, _DE), jnp.float32),
            [pltpu.SemaphoreType.DMA for _ in range(_NB)],
            [pltpu.SemaphoreType.DMA for _ in range(_NB)],
        ]

    mesh = plsc.VectorSubcoreMesh(core_axis_name="c", subcore_axis_name="s")

    def body(*refs):
        if with_ea:
            (x_hbm, src_hbm, dst_hbm, ea_hbm, zx_hbm, zea_hbm,
             acc_out, ea_out,
             sidx_v, didx_v, rows_v, acc_sp, gsem, ssem,
             ea_v, ea_sp, esem, s2sem) = refs
        else:
            (x_hbm, src_hbm, dst_hbm, zx_hbm,
             acc_out,
             sidx_v, didx_v, rows_v, acc_sp, gsem, ssem) = refs

        cid = lax.axis_index("c")
        sid = lax.axis_index("s")
        wid = sid * _NC + cid

        pltpu.sync_copy(zx_hbm, acc_sp.at[pl.ds(sid * _RPT, _RPT)])
        if with_ea:
            pltpu.sync_copy(zea_hbm, ea_sp.at[pl.ds(sid * _RPT, _RPT)])
        plsc.subcore_barrier()

        def run_block(blk):
            pltpu.sync_copy(src_hbm.at[wid, pl.ds(blk * _BLK, _BLK)], sidx_v)
            pltpu.sync_copy(dst_hbm.at[wid, pl.ds(blk * _BLK, _BLK)], didx_v)

            def issue(r, b):
                pltpu.async_copy(x_hbm.at[sidx_v.at[r]], rows_v[b], gsem[b])
                if with_ea:
                    pltpu.async_copy(ea_hbm.at[wid, blk * _BLK + r],
                                     ea_v[b], esem[b])

            def wait_scatter(b):
                pltpu.make_async_copy(rows_v[b], acc_sp.at[didx_v.at[0]],
                                      ssem[b]).wait()
                if with_ea:
                    pltpu.make_async_copy(ea_v[b], ea_sp.at[didx_v.at[0]],
                                          s2sem[b]).wait()

            for b in range(2):
                issue(b, b)

            def outer(rr, carry):
                for b in range(_NB):
                    r = rr * _NB + b
                    bn = (b + 2) % _NB

                    @pl.when(r >= 2)
                    def _():
                        wait_scatter(bn)

                    @pl.when(r + 2 < _BLK)
                    def _():
                        issue(r + 2, bn)

                    pltpu.make_async_copy(
                        x_hbm.at[sidx_v.at[r]], rows_v[b], gsem[b]).wait()
                    pltpu.async_copy(rows_v[b], acc_sp.at[didx_v.at[r]],
                                     ssem[b], add=True)
                    if with_ea:
                        pltpu.make_async_copy(
                            ea_hbm.at[wid, blk * _BLK + r],
                            ea_v[b], esem[b]).wait()
                        pltpu.async_copy(ea_v[b], ea_sp.at[didx_v.at[r]],
                                         s2sem[b], add=True)
                return carry

            lax.fori_loop(0, _BLK // _NB, outer, 0)
            wait_scatter((_BLK - 2) % _NB)
            wait_scatter((_BLK - 1) % _NB)

        for blk in range(_NBLK):
            run_block(blk)
        plsc.subcore_barrier()

        pltpu.sync_copy(acc_sp.at[pl.ds(sid * _RPT, _RPT)],
                        acc_out.at[cid, pl.ds(sid * _RPT, _RPT)])
        if with_ea:
            pltpu.sync_copy(ea_sp.at[pl.ds(sid * _RPT, _RPT)],
                            ea_out.at[cid, pl.ds(sid * _RPT, _RPT)])

    return pl.kernel(
        body, mesh=mesh, out_type=out_type, scratch_types=scratch,
        compiler_params=pltpu.CompilerParams(use_tc_tiling_on_sc=False))


@functools.cache
def _sc_pass(with_ea):
    return _sc_edge_pass(with_ea)


def _onehot_and_invcnt(batch2d):
    gids = lax.broadcasted_iota(jnp.int32, (1, _G), 1)
    onehot = (batch2d == gids).astype(jnp.float32)
    ones = jnp.ones((_N, 1), jnp.float32)
    cnt = lax.dot_general(onehot, ones, (((0,), (0,)), ((), ())))
    return onehot, 1.0 / jnp.maximum(cnt, 1.0)


def _tc_conv1_body(x_ref, s1_ref, batch_ref, w1_ref, be_ref,
                   out_ref, outb_ref):
    x = x_ref[...]
    out = x + s1_ref[0, :_N] + s1_ref[1, :_N]
    onehot, invcnt = _onehot_and_invcnt(batch_ref[...])
    pooled = lax.dot_general(onehot, out, (((0,), (0,)), ((), ())))
    vmsg = pooled * invcnt
    out = out + lax.dot_general(onehot, vmsg, (((1,), (0,)), ((), ())))
    out = jnp.maximum(lax.dot_general(out, w1_ref[...],
                                      (((1,), (0,)), ((), ()))), 0.0)
    x1 = out + x
    out_ref[...] = x1
    outb_ref[...] = x1 + be_ref[...]


_tc_conv1 = pl.pallas_call(
    _tc_conv1_body,
    out_shape=[jax.ShapeDtypeStruct((_N, _D), jnp.float32),
               jax.ShapeDtypeStruct((_N, _D), jnp.float32)],
)


def _tc_conv2_body(x1_ref, s2_ref, ea_ref, batch_ref, we_ref,
                   wn1_ref, bn1_ref, wn2_ref, bn2_ref, wfc_ref, bfc_ref,
                   out_ref):
    x1 = x1_ref[...]
    ea = ea_ref[0, :_N] + ea_ref[1, :_N]
    agg = s2_ref[0, :_N] + s2_ref[1, :_N] + lax.dot_general(
        ea, we_ref[...], (((1,), (0,)), ((), ())))
    h = jnp.maximum(lax.dot_general(agg, wn1_ref[...],
                                    (((1,), (0,)), ((), ()))) + bn1_ref[...],
                    0.0)
    out2 = lax.dot_general(h, wn2_ref[...],
                           (((1,), (0,)), ((), ()))) + bn2_ref[...]
    x2 = out2 + x1
    onehot, invcnt = _onehot_and_invcnt(batch_ref[...])
    pooled = lax.dot_general(onehot, x2, (((0,), (0,)), ((), ()))) * invcnt
    out_ref[...] = lax.dot_general(pooled, wfc_ref[...],
                                   (((1,), (0,)), ((), ()))) + bfc_ref[...]


_tc_conv2 = pl.pallas_call(
    _tc_conv2_body,
    out_shape=jax.ShapeDtypeStruct((_G, _D), jnp.float32),
)


def kernel(x, edge_index, edge_attr, batch, W1, We, be, Wn1, bn1, Wn2, bn2,
           Wfc, bfc):
    src = edge_index[0].reshape(_NW, _NCHUNK, _CHUNK)
    dst = edge_index[1].reshape(_NW, _NCHUNK, _CHUNK)
    ea = edge_attr.reshape(_NW, _NCHUNK, _CHUNK, _DE)
    zx = jnp.zeros((_RPT, _D), jnp.float32)
    zea = jnp.zeros((_RPT, _DE), jnp.float32)
    batch2d = batch[:, None]

    s1p = _sc_pass(False)(x, src, dst, zx)
    if isinstance(s1p, (list, tuple)):
        (s1p,) = s1p
    x1, x1b = _tc_conv1(x, s1p, batch2d, W1, be[None, :])
    s2p, eap = _sc_pass(True)(x1b, src, dst, ea, zx, zea)
    return _tc_conv2(x1, s2p, eap, batch2d, We,
                     Wn1, bn1[None, :], Wn2, bn2[None, :], Wfc, bfc[None, :])


# speedup vs baseline: 1.4535x; 1.3604x over previous
import functools

import jax
import jax.numpy as jnp
from jax import lax
from jax.experimental import pallas as pl
from jax.experimental.pallas import tpu as pltpu
from jax.experimental.pallas import tpu_sc as plsc

_N = 10000
_E = 320000
_D = 128
_DE = 16
_G = 64

_NC = 2
_NS = 16
_NW = _NC * _NS
_EPW = _E // _NW
_CHUNK = 80
_NCHUNK = _EPW // _CHUNK
_NP = 10112
_RPT = _NP // _NS


def _sc_edge_pass(with_ea):
    out_type = [jax.ShapeDtypeStruct((_NC, _NP, _D), jnp.float32)]
    scratch = [
        [pltpu.VMEM((_CHUNK,), jnp.int32) for _ in range(4)],
        [pltpu.VMEM((_CHUNK,), jnp.int32) for _ in range(4)],
        [pltpu.VMEM((_CHUNK, _D), jnp.float32) for _ in range(2)],
        pltpu.VMEM_SHARED((_NP, _D), jnp.float32),
        [pltpu.SemaphoreType.DMA for _ in range(4)],
        [pltpu.SemaphoreType.DMA for _ in range(2)],
    ]
    if with_ea:
        out_type.append(jax.ShapeDtypeStruct((_NC, _NP, _DE), jnp.float32))
        scratch += [
            [pltpu.VMEM((_DE, _CHUNK), jnp.float32) for _ in range(2)],
            [pltpu.VMEM((_CHUNK, _DE), jnp.float32) for _ in range(2)],
            pltpu.VMEM_SHARED((_NP, _DE), jnp.float32),
            [pltpu.SemaphoreType.DMA for _ in range(2)],
        ]

    mesh = plsc.VectorSubcoreMesh(core_axis_name="c", subcore_axis_name="s")

    def body(*refs):
        if with_ea:
            (x_hbm, src_hbm, dst_hbm, eat_hbm, zx_hbm, zea_hbm,
             acc_out, ea_out,
             sidx, didx, rows_v, acc_sp, isem, gsem,
             eas_v, ea_v, ea_sp, esem) = refs
        else:
            (x_hbm, src_hbm, dst_hbm, zx_hbm,
             acc_out,
             sidx, didx, rows_v, acc_sp, isem, gsem) = refs

        cid = lax.axis_index("c")
        sid = lax.axis_index("s")
        wid = sid * _NC + cid
        base = wid * _EPW

        pltpu.sync_copy(zx_hbm, acc_sp.at[pl.ds(sid * _RPT, _RPT)])
        if with_ea:
            pltpu.sync_copy(zea_hbm, ea_sp.at[pl.ds(sid * _RPT, _RPT)])
        plsc.subcore_barrier()

        def idx_cp(j, s):
            off = pl.multiple_of(base + j * _CHUNK, 8)
            return (pltpu.make_async_copy(
                        src_hbm.at[pl.ds(off, _CHUNK)], sidx[s], isem[s]),
                    pltpu.make_async_copy(
                        dst_hbm.at[pl.ds(off, _CHUNK)], didx[s], isem[s]))

        def stage_idx(j, s):
            for c in idx_cp(j, s):
                c.start()

        def wait_idx(j, s):
            for c in idx_cp(j, s):
                c.wait()

        def ea_cp(j, s):
            off = pl.multiple_of(base + j * _CHUNK, 8)
            return pltpu.make_async_copy(
                eat_hbm.at[:, pl.ds(off, _CHUNK)], eas_v[s], esem[s])

        def issue_gather(j, s, b):
            pltpu.async_copy(x_hbm.at[sidx[s]], rows_v[b], gsem[b])

        iota16 = lax.iota(jnp.int32, 16)

        def transpose_ea(s, b):
            def tbody(r, carry):
                v = plsc.load_gather(eas_v[s],
                                     [iota16, jnp.full((16,), r, jnp.int32)])
                ea_v[b][r, :] = v
                return carry
            lax.fori_loop(0, _CHUNK, tbody, 0)

        stage_idx(0, 0)
        stage_idx(1, 1)
        if with_ea:
            ea_cp(0, 0).start()
            ea_cp(1, 1).start()
        wait_idx(0, 0)
        wait_idx(1, 1)
        issue_gather(0, 0, 0)
        issue_gather(1, 1, 1)
        stage_idx(2, 2)
        stage_idx(3, 3)

        def step(j, k, b):
            pltpu.make_async_copy(x_hbm.at[sidx[k]], rows_v[b],
                                  gsem[b]).wait()
            if with_ea:
                ea_cp(j, b).wait()
                transpose_ea(b, b)
            pltpu.sync_copy(rows_v[b], acc_sp.at[didx[k]], add=True)
            if with_ea:
                pltpu.sync_copy(ea_v[b], ea_sp.at[didx[k]], add=True)

                @pl.when(j + 2 < _NCHUNK)
                def _():
                    ea_cp(j + 2, b).start()

            @pl.when(j + 2 < _NCHUNK)
            def _():
                wait_idx(j + 2, (k + 2) % 4)
                issue_gather(j + 2, (k + 2) % 4, b)

            @pl.when(j + 4 < _NCHUNK)
            def _():
                stage_idx(j + 4, k)

        def outer(jj, carry):
            for k in range(4):
                step(jj * 4 + k, k, k % 2)
            return carry

        lax.fori_loop(0, _NCHUNK // 4, outer, 0)
        step(_NCHUNK - 1, (_NCHUNK - 1) % 4, (_NCHUNK - 1) % 2)
        plsc.subcore_barrier()

        pltpu.sync_copy(acc_sp.at[pl.ds(sid * _RPT, _RPT)],
                        acc_out.at[cid, pl.ds(sid * _RPT, _RPT)])
        if with_ea:
            pltpu.sync_copy(ea_sp.at[pl.ds(sid * _RPT, _RPT)],
                            ea_out.at[cid, pl.ds(sid * _RPT, _RPT)])

    return pl.kernel(
        body, mesh=mesh, out_type=out_type, scratch_types=scratch,
        compiler_params=pltpu.CompilerParams(use_tc_tiling_on_sc=False,
                                             needs_layout_passes=False))


@functools.cache
def _sc_pass(with_ea):
    return _sc_edge_pass(with_ea)


def _onehot_and_invcnt(batch2d):
    gids = lax.broadcasted_iota(jnp.int32, (1, _G), 1)
    onehot = (batch2d == gids).astype(jnp.float32)
    ones = jnp.ones((_N, 1), jnp.float32)
    cnt = lax.dot_general(onehot, ones, (((0,), (0,)), ((), ())))
    return onehot, 1.0 / jnp.maximum(cnt, 1.0)


def _tc_conv1_body(x_ref, s1_ref, batch_ref, w1_ref, be_ref,
                   out_ref, outb_ref):
    x = x_ref[...]
    out = x + s1_ref[0, :_N] + s1_ref[1, :_N]
    onehot, invcnt = _onehot_and_invcnt(batch_ref[...])
    pooled = lax.dot_general(onehot, out, (((0,), (0,)), ((), ())))
    vmsg = pooled * invcnt
    out = out + lax.dot_general(onehot, vmsg, (((1,), (0,)), ((), ())))
    out = jnp.maximum(lax.dot_general(out, w1_ref[...],
                                      (((1,), (0,)), ((), ()))), 0.0)
    x1 = out + x
    out_ref[...] = x1
    outb_ref[...] = x1 + be_ref[...]


_tc_conv1 = pl.pallas_call(
    _tc_conv1_body,
    out_shape=[jax.ShapeDtypeStruct((_N, _D), jnp.float32),
               jax.ShapeDtypeStruct((_N, _D), jnp.float32)],
)


def _tc_conv2_body(x1_ref, s2_ref, ea_ref, batch_ref, we_ref,
                   wn1_ref, bn1_ref, wn2_ref, bn2_ref, wfc_ref, bfc_ref,
                   out_ref):
    x1 = x1_ref[...]
    ea = ea_ref[0, :_N] + ea_ref[1, :_N]
    agg = s2_ref[0, :_N] + s2_ref[1, :_N] + lax.dot_general(
        ea, we_ref[...], (((1,), (0,)), ((), ())))
    h = jnp.maximum(lax.dot_general(agg, wn1_ref[...],
                                    (((1,), (0,)), ((), ()))) + bn1_ref[...],
                    0.0)
    out2 = lax.dot_general(h, wn2_ref[...],
                           (((1,), (0,)), ((), ()))) + bn2_ref[...]
    x2 = out2 + x1
    onehot, invcnt = _onehot_and_invcnt(batch_ref[...])
    pooled = lax.dot_general(onehot, x2, (((0,), (0,)), ((), ()))) * invcnt
    out_ref[...] = lax.dot_general(pooled, wfc_ref[...],
                                   (((1,), (0,)), ((), ()))) + bfc_ref[...]


_tc_conv2 = pl.pallas_call(
    _tc_conv2_body,
    out_shape=jax.ShapeDtypeStruct((_G, _D), jnp.float32),
)


def kernel(x, edge_index, edge_attr, batch, W1, We, be, Wn1, bn1, Wn2, bn2,
           Wfc, bfc):
    src = edge_index[0]
    dst = edge_index[1]
    eat = edge_attr.T
    zx = jnp.zeros((_RPT, _D), jnp.float32)
    zea = jnp.zeros((_RPT, _DE), jnp.float32)
    batch2d = batch[:, None]

    s1p, eap = _sc_pass(True)(x, src, dst, eat, zx, zea)
    x1, x1b = _tc_conv1(x, s1p, batch2d, W1, be[None, :])
    s2p = _sc_pass(False)(x1b, src, dst, zx)
    if isinstance(s2p, (list, tuple)):
        (s2p,) = s2p
    return _tc_conv2(x1, s2p, eap, batch2d, We,
                     Wn1, bn1[None, :], Wn2, bn2[None, :], Wfc, bfc[None, :])


# speedup vs baseline: 1.6109x; 1.1083x over previous
import functools

import jax
import jax.numpy as jnp
from jax import lax
from jax.experimental import pallas as pl
from jax.experimental.pallas import tpu as pltpu
from jax.experimental.pallas import tpu_sc as plsc

_N = 10000
_E = 320000
_D = 128
_DE = 16
_G = 64

_NC = 2
_NS = 16
_NW = _NC * _NS
_EPW = _E // _NW
_CHUNK = 80
_NCHUNK = _EPW // _CHUNK
_NP = 10112
_RPT = _NP // _NS


def _sc_edge_pass(with_ea):
    out_type = [jax.ShapeDtypeStruct((_NC, _NP, _D), jnp.float32)]
    scratch = [
        [pltpu.VMEM((_CHUNK,), jnp.int32) for _ in range(4)],
        [pltpu.VMEM((_CHUNK,), jnp.int32) for _ in range(4)],
        [pltpu.VMEM((_CHUNK, _D), jnp.float32) for _ in range(2)],
        pltpu.VMEM_SHARED((_NP, _D), jnp.float32),
        [pltpu.SemaphoreType.DMA for _ in range(4)],
        [pltpu.SemaphoreType.DMA for _ in range(2)],
        [pltpu.SemaphoreType.DMA for _ in range(2)],
    ]
    if with_ea:
        out_type.append(jax.ShapeDtypeStruct((_NC, _NP, _DE), jnp.float32))
        scratch += [
            [pltpu.VMEM((_DE, _CHUNK), jnp.float32) for _ in range(2)],
            [pltpu.VMEM((_CHUNK, _DE), jnp.float32) for _ in range(2)],
            pltpu.VMEM_SHARED((_NP, _DE), jnp.float32),
            [pltpu.SemaphoreType.DMA for _ in range(2)],
        ]

    mesh = plsc.VectorSubcoreMesh(core_axis_name="c", subcore_axis_name="s")

    def body(*refs):
        if with_ea:
            (x_hbm, ei_hbm, eat_hbm, zx_hbm, zea_hbm,
             acc_out, ea_out,
             sidx, didx, rows_v, acc_sp, isem, gsem, ssem,
             eas_v, ea_v, ea_sp, esem) = refs
        else:
            (x_hbm, ei_hbm, zx_hbm,
             acc_out,
             sidx, didx, rows_v, acc_sp, isem, gsem, ssem) = refs

        cid = lax.axis_index("c")
        sid = lax.axis_index("s")
        wid = sid * _NC + cid
        base = wid * _EPW

        pltpu.sync_copy(zx_hbm, acc_sp.at[pl.ds(sid * _RPT, _RPT)])
        if with_ea:
            pltpu.sync_copy(zea_hbm, ea_sp.at[pl.ds(sid * _RPT, _RPT)])
        plsc.subcore_barrier()

        def idx_cp(j, s):
            off = pl.multiple_of(base + j * _CHUNK, 8)
            return (pltpu.make_async_copy(
                        ei_hbm.at[0, pl.ds(off, _CHUNK)], sidx[s], isem[s]),
                    pltpu.make_async_copy(
                        ei_hbm.at[1, pl.ds(off, _CHUNK)], didx[s], isem[s]))

        def stage_idx(j, s):
            for c in idx_cp(j, s):
                c.start()

        def wait_idx(j, s):
            for c in idx_cp(j, s):
                c.wait()

        def ea_cp(j, s):
            off = pl.multiple_of(base + j * _CHUNK, 8)
            return pltpu.make_async_copy(
                eat_hbm.at[:, pl.ds(off, _CHUNK)], eas_v[s], esem[s])

        def issue_gather(j, s, b):
            pltpu.async_copy(x_hbm.at[sidx[s]], rows_v[b], gsem[b])

        iota16 = lax.iota(jnp.int32, 16)

        def transpose_ea(s, b):
            def tbody(r, carry):
                v = plsc.load_gather(eas_v[s],
                                     [iota16, jnp.full((16,), r, jnp.int32)])
                ea_v[b][r, :] = v
                return carry
            lax.fori_loop(0, _CHUNK, tbody, 0)

        stage_idx(0, 0)
        stage_idx(1, 1)
        if with_ea:
            ea_cp(0, 0).start()
            ea_cp(1, 1).start()
        wait_idx(0, 0)
        wait_idx(1, 1)
        issue_gather(0, 0, 0)
        issue_gather(1, 1, 1)
        stage_idx(2, 2)
        stage_idx(3, 3)

        def step(j, k, b):
            pltpu.make_async_copy(x_hbm.at[sidx[k]], rows_v[b],
                                  gsem[b]).wait()
            pltpu.async_copy(rows_v[b], acc_sp.at[didx[k]], ssem[b],
                             add=True)
            if with_ea:
                ea_cp(j, b).wait()
                transpose_ea(b, b)
                pltpu.sync_copy(ea_v[b], ea_sp.at[didx[k]], add=True)

                @pl.when(j + 2 < _NCHUNK)
                def _():
                    ea_cp(j + 2, b).start()
            pltpu.make_async_copy(rows_v[b], acc_sp.at[didx[k]],
                                  ssem[b]).wait()

            @pl.when(j + 2 < _NCHUNK)
            def _():
                wait_idx(j + 2, (k + 2) % 4)
                issue_gather(j + 2, (k + 2) % 4, b)

            @pl.when(j + 4 < _NCHUNK)
            def _():
                stage_idx(j + 4, k)

        def outer(jj, carry):
            for k in range(4):
                step(jj * 4 + k, k, k % 2)
            return carry

        lax.fori_loop(0, _NCHUNK // 4, outer, 0)
        step(_NCHUNK - 1, (_NCHUNK - 1) % 4, (_NCHUNK - 1) % 2)
        plsc.subcore_barrier()

        pltpu.sync_copy(acc_sp.at[pl.ds(sid * _RPT, _RPT)],
                        acc_out.at[cid, pl.ds(sid * _RPT, _RPT)])
        if with_ea:
            pltpu.sync_copy(ea_sp.at[pl.ds(sid * _RPT, _RPT)],
                            ea_out.at[cid, pl.ds(sid * _RPT, _RPT)])

    return pl.kernel(
        body, mesh=mesh, out_type=out_type, scratch_types=scratch,
        compiler_params=pltpu.CompilerParams(use_tc_tiling_on_sc=False,
                                             needs_layout_passes=False))


@functools.cache
def _sc_pass(with_ea):
    return _sc_edge_pass(with_ea)


def _onehot_and_invcnt(batch2d):
    gids = lax.broadcasted_iota(jnp.int32, (1, _G), 1)
    onehot = (batch2d == gids).astype(jnp.float32)
    ones = jnp.ones((_N, 1), jnp.float32)
    cnt = lax.dot_general(onehot, ones, (((0,), (0,)), ((), ())))
    return onehot, 1.0 / jnp.maximum(cnt, 1.0)


def _tc_conv1_body(x_ref, s1_ref, batch_ref, w1_ref, be_ref,
                   out_ref, outb_ref):
    x = x_ref[...]
    out = x + s1_ref[0, :_N] + s1_ref[1, :_N]
    onehot, invcnt = _onehot_and_invcnt(batch_ref[...])
    pooled = lax.dot_general(onehot, out, (((0,), (0,)), ((), ())))
    vmsg = pooled * invcnt
    out = out + lax.dot_general(onehot, vmsg, (((1,), (0,)), ((), ())))
    out = jnp.maximum(lax.dot_general(out, w1_ref[...],
                                      (((1,), (0,)), ((), ()))), 0.0)
    x1 = out + x
    out_ref[...] = x1
    outb_ref[...] = x1 + be_ref[...]


_tc_conv1 = pl.pallas_call(
    _tc_conv1_body,
    out_shape=[jax.ShapeDtypeStruct((_N, _D), jnp.float32),
               jax.ShapeDtypeStruct((_N, _D), jnp.float32)],
)


def _tc_conv2_body(x1_ref, s2_ref, ea_ref, batch_ref, we_ref,
                   wn1_ref, bn1_ref, wn2_ref, bn2_ref, wfc_ref, bfc_ref,
                   out_ref):
    x1 = x1_ref[...]
    ea = ea_ref[0, :_N] + ea_ref[1, :_N]
    agg = s2_ref[0, :_N] + s2_ref[1, :_N] + lax.dot_general(
        ea, we_ref[...], (((1,), (0,)), ((), ())))
    h = jnp.maximum(lax.dot_general(agg, wn1_ref[...],
                                    (((1,), (0,)), ((), ()))) + bn1_ref[...],
                    0.0)
    out2 = lax.dot_general(h, wn2_ref[...],
                           (((1,), (0,)), ((), ()))) + bn2_ref[...]
    x2 = out2 + x1
    onehot, invcnt = _onehot_and_invcnt(batch_ref[...])
    pooled = lax.dot_general(onehot, x2, (((0,), (0,)), ((), ()))) * invcnt
    out_ref[...] = lax.dot_general(pooled, wfc_ref[...],
                                   (((1,), (0,)), ((), ()))) + bfc_ref[...]


_tc_conv2 = pl.pallas_call(
    _tc_conv2_body,
    out_shape=jax.ShapeDtypeStruct((_G, _D), jnp.float32),
)


def kernel(x, edge_index, edge_attr, batch, W1, We, be, Wn1, bn1, Wn2, bn2,
           Wfc, bfc):
    eat = edge_attr.T
    zx = jnp.zeros((_RPT, _D), jnp.float32)
    zea = jnp.zeros((_RPT, _DE), jnp.float32)
    batch2d = batch[:, None]

    s1p, eap = _sc_pass(True)(x, edge_index, eat, zx, zea)
    x1, x1b = _tc_conv1(x, s1p, batch2d, W1, be[None, :])
    s2p = _sc_pass(False)(x1b, edge_index, zx)
    if isinstance(s2p, (list, tuple)):
        (s2p,) = s2p
    return _tc_conv2(x1, s2p, eap, batch2d, We,
                     Wn1, bn1[None, :], Wn2, bn2[None, :], Wfc, bfc[None, :])
